# baseline (speedup 1.0000x reference)
import functools

import jax
import jax.numpy as jnp
from jax import lax
from jax.experimental import pallas as pl
from jax.experimental.pallas import tpu as pltpu
from jax.experimental.pallas import tpu_sc as plsc

N_NODES = 10000
N_EDGES = 160000
D_FEAT = 128
D_EDGE = 16
HID = 32
N_GRAPHS = 64
N_PAIRS = 128
N_OUT = 32

ROW_BLK = 1000
N_BLKS = N_NODES // ROW_BLK

NC = 2
NS = 16
NW = NC * NS
CHUNK = 128
N_CHUNKS = N_EDGES // CHUNK
CHUNKS_PER_W = -(-N_CHUNKS // NW)
N_PAD = 10240
N_PASS = 4
PASS_R = N_PAD // N_PASS

_BF = jnp.bfloat16
_F32 = jnp.float32


def _node1_body(x_ref, t_ref, wr_ref, xt_ref, root_ref):
    xb = x_ref[...].astype(_BF)
    xt_ref[...] = jnp.dot(xb, t_ref[...], preferred_element_type=_F32)
    root_ref[...] = jnp.dot(xb, wr_ref[...], preferred_element_type=_F32)


def _node_transform1(x, t1p, wr1):
    return pl.pallas_call(
        _node1_body,
        grid=(N_BLKS,),
        in_specs=[
            pl.BlockSpec((ROW_BLK, D_FEAT), lambda b: (b, 0)),
            pl.BlockSpec((D_FEAT, D_EDGE * HID), lambda b: (0, 0)),
            pl.BlockSpec((D_FEAT, HID), lambda b: (0, 0)),
        ],
        out_specs=[
            pl.BlockSpec((ROW_BLK, D_EDGE * HID), lambda b: (b, 0)),
            pl.BlockSpec((ROW_BLK, HID), lambda b: (b, 0)),
        ],
        out_shape=[
            jax.ShapeDtypeStruct((N_NODES, D_EDGE * HID), _F32),
            jax.ShapeDtypeStruct((N_NODES, HID), _F32),
        ],
    )(x, t1p, wr1)


def _node2_body(agg_ref, root_ref, t_ref, wr_ref, xt_ref, root2_ref):
    h = jnp.maximum(jnp.sum(agg_ref[...], axis=0) + root_ref[...], 0.0)
    hb = h.astype(_BF)
    xt_ref[...] = jnp.dot(hb, t_ref[...], preferred_element_type=_F32)
    root2_ref[...] = jnp.dot(hb, wr_ref[...], preferred_element_type=_F32)


def _node_transform2(agg, root1, t2p, wr2):
    return pl.pallas_call(
        _node2_body,
        grid=(N_BLKS,),
        in_specs=[
            pl.BlockSpec((NW, ROW_BLK, HID), lambda b: (0, b, 0)),
            pl.BlockSpec((ROW_BLK, HID), lambda b: (b, 0)),
            pl.BlockSpec((HID, D_EDGE * HID), lambda b: (0, 0)),
            pl.BlockSpec((HID, HID), lambda b: (0, 0)),
        ],
        out_specs=[
            pl.BlockSpec((ROW_BLK, D_EDGE * HID), lambda b: (b, 0)),
            pl.BlockSpec((ROW_BLK, HID), lambda b: (b, 0)),
        ],
        out_shape=[
            jax.ShapeDtypeStruct((N_NODES, D_EDGE * HID), _F32),
            jax.ShapeDtypeStruct((N_NODES, HID), _F32),
        ],
    )(agg, root1, t2p, wr2)


def _head_body(agg_ref, root_ref, i_ref, wu_ref, ia_ref, ib_ref, out_ref,
               pooled_acc):
    b = pl.program_id(0)

    @pl.when(b == 0)
    def _():
        pooled_acc[...] = jnp.zeros_like(pooled_acc)

    h2 = jnp.trunc(jnp.maximum(jnp.sum(agg_ref[...], axis=0)
                               + root_ref[...], 0.0))
    gids = i_ref[0]
    mask = (lax.broadcasted_iota(jnp.int32, (N_GRAPHS, ROW_BLK), 0)
            == gids).astype(_F32)
    pooled_acc[...] += jnp.dot(mask, h2, preferred_element_type=_F32,
                               precision=lax.Precision.HIGHEST)

    @pl.when(b == N_BLKS - 1)
    def _():
        u = jnp.maximum(
            jnp.dot(pooled_acc[...].astype(_BF), wu_ref[...],
                    preferred_element_type=_F32), 0.0)
        giota = lax.broadcasted_iota(jnp.int32, (N_PAIRS, N_GRAPHS), 1)
        sel = ((giota == ib_ref[...]).astype(_F32)
               - (giota == ia_ref[...]).astype(_F32))
        out_ref[...] = jnp.dot(sel, u, preferred_element_type=_F32,
                               precision=lax.Precision.HIGHEST)


def _pool_head(agg, root2, i3d, wu, ia, ib):
    return pl.pallas_call(
        _head_body,
        grid=(N_BLKS,),
        in_specs=[
            pl.BlockSpec((NW, ROW_BLK, HID), lambda b: (0, b, 0)),
            pl.BlockSpec((ROW_BLK, HID), lambda b: (b, 0)),
            pl.BlockSpec((1, 1, ROW_BLK), lambda b: (b, 0, 0)),
            pl.BlockSpec((HID, N_OUT), lambda b: (0, 0)),
            pl.BlockSpec((N_PAIRS, 1), lambda b: (0, 0)),
            pl.BlockSpec((N_PAIRS, 1), lambda b: (0, 0)),
        ],
        out_specs=pl.BlockSpec((N_PAIRS, N_OUT), lambda b: (0, 0)),
        out_shape=jax.ShapeDtypeStruct((N_PAIRS, N_OUT), _F32),
        scratch_shapes=[pltpu.VMEM((N_GRAPHS, HID), _F32)],
    )(agg, root2, i3d, wu, ia, ib)


def _edge_body(xt_hbm, src_hbm, ef_hbm, msg_hbm,
               src_v, e_v, rows_v, msg_v, sem):
    c = lax.axis_index("c")
    s = lax.axis_index("s")
    w = s * NC + c

    def chunk_body(j, carry):
        cid = w + j * NW

        @pl.when(cid < N_CHUNKS)
        def _():
            base = cid * CHUNK
            pltpu.sync_copy(src_hbm.at[pl.ds(base, CHUNK)], src_v)
            pltpu.sync_copy(ef_hbm.at[pl.ds(base * D_EDGE, CHUNK * D_EDGE)],
                            e_v)
            pltpu.async_copy(xt_hbm.at[src_v], rows_v, sem).wait()

            def edge_loop(ei, inner):
                ew = e_v[pl.ds(ei * D_EDGE, 16)]
                acc0 = jnp.zeros((16,), _F32)
                acc1 = jnp.zeros((16,), _F32)
                for k in range(D_EDGE):
                    wk = ew[k]
                    acc0 = acc0 + wk * rows_v[ei, pl.ds(k * HID, 16)]
                    acc1 = acc1 + wk * rows_v[ei, pl.ds(k * HID + 16, 16)]
                msg_v[ei, pl.ds(0, 16)] = acc0
                msg_v[ei, pl.ds(16, 16)] = acc1
                return inner

            lax.fori_loop(0, CHUNK, edge_loop, 0)
            pltpu.sync_copy(msg_v, msg_hbm.at[pl.ds(base, CHUNK)])

        return carry

    lax.fori_loop(0, CHUNKS_PER_W, chunk_body, 0)


@functools.partial(jax.jit, static_argnames=())
def _edge_stage(xt, src, ef):
    mesh = plsc.VectorSubcoreMesh(core_axis_name="c", subcore_axis_name="s")
    f = pl.kernel(
        _edge_body,
        out_type=jax.ShapeDtypeStruct((N_EDGES, HID), _F32),
        mesh=mesh,
        scratch_types=[
            pltpu.VMEM((CHUNK,), jnp.int32),
            pltpu.VMEM((CHUNK * D_EDGE,), _F32),
            pltpu.VMEM((CHUNK, D_EDGE * HID), _F32),
            pltpu.VMEM((CHUNK, HID), _F32),
            pltpu.SemaphoreType.DMA,
        ],
    )
    return f(xt, src, ef)


def _agg_body(msg_hbm, tgt_hbm, out_hbm, tgt_v, msg_v, agg_v, sem):
    c = lax.axis_index("c")
    s = lax.axis_index("s")
    w = s * NC + c

    def pass_body(p, carry):
        plo = p * PASS_R
        z16 = jnp.zeros((16,), _F32)

        def zi(ii, cc):
            agg_v[pl.ds(ii * 16, 16)] = z16
            return cc

        lax.fori_loop(0, (PASS_R + 8) * HID // 16, zi, 0)

        def chunk_body(j, cc):
            cid = w + j * NW

            @pl.when(cid < N_CHUNKS)
            def _():
                base = cid * CHUNK
                pltpu.sync_copy(tgt_hbm.at[pl.ds(base, CHUNK)], tgt_v)
                pltpu.sync_copy(
                    msg_hbm.at[pl.ds(base * HID, CHUNK * HID)], msg_v)

                def group(g, cc2):
                    tv = tgt_v[pl.ds(g * 16, 16)]
                    rel = tv - plo
                    for l in range(16):
                        rl = rel[l]
                        ok = (rl >= 0) & (rl < PASS_R)
                        row = jnp.where(ok, rl, PASS_R)
                        moff = (g * 16 + l) * HID
                        agg_v[pl.ds(row * HID, 16)] = (
                            agg_v[pl.ds(row * HID, 16)]
                            + msg_v[pl.ds(moff, 16)])
                        agg_v[pl.ds(row * HID + 16, 16)] = (
                            agg_v[pl.ds(row * HID + 16, 16)]
                            + msg_v[pl.ds(moff + 16, 16)])
                    return cc2

                lax.fori_loop(0, CHUNK // 16, group, 0)

            return cc

        lax.fori_loop(0, CHUNKS_PER_W, chunk_body, 0)
        pltpu.sync_copy(agg_v.at[pl.ds(0, PASS_R * HID)],
                        out_hbm.at[w].at[pl.ds(plo * HID, PASS_R * HID)])
        return carry

    lax.fori_loop(0, N_PASS, pass_body, 0)


@functools.partial(jax.jit, static_argnames=())
def _agg_stage(msg, tgt):
    mesh = plsc.VectorSubcoreMesh(core_axis_name="c", subcore_axis_name="s")
    f = pl.kernel(
        _agg_body,
        out_type=jax.ShapeDtypeStruct((NW, N_PAD * HID), _F32),
        mesh=mesh,
        scratch_types=[
            pltpu.VMEM((CHUNK,), jnp.int32),
            pltpu.VMEM((CHUNK * HID,), _F32),
            pltpu.VMEM(((PASS_R + 8) * HID,), _F32),
            pltpu.SemaphoreType.DMA,
        ],
    )
    return f(msg.reshape(-1), tgt).reshape(NW, N_PAD, HID)


def kernel(x, edge_index, e, i, idx_a, idx_b, W_k1, b_k1, W_root1, b_root1,
           W_k2, b_k2, W_root2, b_root2, W_u, b_u):
    src = edge_index[0]
    tgt = edge_index[1]

    t1p = (W_k1.reshape(D_EDGE, D_FEAT, HID).transpose(1, 0, 2)
           .reshape(D_FEAT, D_EDGE * HID).astype(_BF))
    t2p = (W_k2.reshape(D_EDGE, HID, HID).transpose(1, 0, 2)
           .reshape(HID, D_EDGE * HID).astype(_BF))
    wr1 = W_root1.astype(_BF)
    wr2 = W_root2.astype(_BF)
    wu = W_u.astype(_BF)
    i3d = i.reshape(N_BLKS, 1, ROW_BLK)
    ia = idx_a.reshape(N_PAIRS, 1)
    ib = idx_b.reshape(N_PAIRS, 1)

    xt1, root1 = _node_transform1(x, t1p, wr1)
    ef = e.reshape(-1)

    agg1 = _agg_stage(_edge_stage(xt1, src, ef), tgt)
    xt2, root2 = _node_transform2(agg1, root1, t2p, wr2)
    agg2 = _agg_stage(_edge_stage(xt2, src, ef), tgt)
    return _pool_head(agg2, root2, i3d, wu, ia, ib)
